# baseline (speedup 1.0000x reference)
import functools
import math

import jax
import jax.numpy as jnp
from jax import lax
from jax.experimental import pallas as pl
from jax.experimental.pallas import tpu as pltpu
from jax.experimental.pallas import tpu_sc as plsc

B = 4
N_SOLUTE = 2076
N_SOLVENT = 16335
NFEAT = 128
NCLASS = 100

NC = 2
NS = 16
L = 16
NW = NC * NS
MACRO = 16
SPMEM_BUDGET = 6 * 2**20


def _mm_body(x_ref, w_ref, b_ref, o_ref):
    o_ref[...] = (
        jnp.dot(x_ref[...], w_ref[...], preferred_element_type=jnp.float32)
        + b_ref[...]
    )


def _matmul_bias(x, W, b, block_rows=512):
    n, din = x.shape
    dout = W.shape[0]
    n_pad = ((n + block_rows - 1) // block_rows) * block_rows
    if n_pad != n:
        x = jnp.pad(x, ((0, n_pad - n), (0, 0)))
    out = pl.pallas_call(
        _mm_body,
        grid=(n_pad // block_rows,),
        in_specs=[
            pl.BlockSpec((block_rows, din), lambda i: (i, 0)),
            pl.BlockSpec((din, dout), lambda i: (0, 0)),
            pl.BlockSpec((1, dout), lambda i: (0, 0)),
        ],
        out_specs=pl.BlockSpec((block_rows, dout), lambda i: (i, 0)),
        out_shape=jax.ShapeDtypeStruct((n_pad, dout), jnp.float32),
    )(x, W.T, b.reshape(1, dout))
    return out[:n]


def _edge_kernel_body(nch, d_pad, F, scale, e_real, nmacro_t, n_pad, *refs):
    (src_hbm, dst_hbm, q_hbm, k_hbm, z_hbm, vt_hbm, out_hbm,
     src_v, dst_v, ex_v, q_rows, k_rows, v_rows, sem1, sem2, agg_sp) = refs

    c = lax.axis_index("c")
    s = lax.axis_index("s")
    wid = s * NC + c
    mb = wid * nmacro_t
    rslice = n_pad // NS
    iota = lax.iota(jnp.int32, L)
    inv = jnp.float32(scale)

    pltpu.sync_copy(src_hbm.at[pl.ds(mb, nmacro_t)], src_v)
    pltpu.sync_copy(dst_hbm.at[pl.ds(mb, nmacro_t)], dst_v)

    def phase1(j, carry):
        d1 = pltpu.async_copy(q_hbm.at[dst_v.at[j]], q_rows, sem1)
        d2 = pltpu.async_copy(k_hbm.at[src_v.at[j]], k_rows, sem2)
        d1.wait()
        d2.wait()
        for qq in range(MACRO // L):
            ev = jnp.zeros((L,), jnp.float32)
            for i16 in range(L):
                i = L * qq + i16
                acc = q_rows[i, pl.ds(0, L)] * k_rows[i, pl.ds(0, L)]
                for w in range(1, d_pad // L):
                    acc = acc + (q_rows[i, pl.ds(L * w, L)]
                                 * k_rows[i, pl.ds(L * w, L)])
                ev = jnp.where(iota == i16, jnp.sum(acc) * inv, ev)
            gid = (mb + j) * MACRO + L * qq + iota
            ex_v[pl.ds(j * MACRO + L * qq, L)] = jnp.where(
                gid < e_real, jnp.exp(ev), 0.0)
        return carry
    lax.fori_loop(0, nmacro_t, phase1, 0)

    def chunk_loop(ci, carry):
        pltpu.sync_copy(z_hbm.at[pl.ds(s * rslice, rslice)],
                        agg_sp.at[pl.ds(s * rslice, rslice)])
        plsc.subcore_barrier()

        def phase2(j, carry2):
            pltpu.async_copy(vt_hbm.at[ci].at[src_v.at[j]],
                             v_rows, sem1).wait()
            for qq in range(MACRO // L):
                ex16 = ex_v[pl.ds(j * MACRO + L * qq, L)]
                for i16 in range(L):
                    i = L * qq + i16
                    exs = ex16[i16]
                    for w in range(F // L):
                        sl = pl.ds(L * w, L)
                        v_rows[i, sl] = v_rows[i, sl] * exs
            pltpu.sync_copy(v_rows, agg_sp.at[dst_v.at[j]], add=True)
            return carry2
        lax.fori_loop(0, nmacro_t, phase2, 0)
        plsc.subcore_barrier()
        pltpu.sync_copy(agg_sp.at[pl.ds(s * rslice, rslice)],
                        out_hbm.at[ci].at[c].at[pl.ds(s * rslice, rslice)])
        return carry
    lax.fori_loop(0, nch, chunk_loop, 0)


def _sc_edge_aggregate(src2, dst2, q, k, vt3, zeros_nf,
                       *, d, e_real, nmacro_t, n_pad):
    nch, _, F = vt3.shape
    d_pad = q.shape[1]
    mesh = plsc.VectorSubcoreMesh(core_axis_name="c", subcore_axis_name="s",
                                  num_cores=NC, num_subcores=NS)
    ept = nmacro_t * MACRO
    body = functools.partial(_edge_kernel_body, nch, d_pad, F,
                             1.0 / math.sqrt(d), e_real, nmacro_t, n_pad)
    fn = pl.kernel(
        body,
        out_type=jax.ShapeDtypeStruct((nch, NC, n_pad, F), jnp.float32),
        mesh=mesh,
        compiler_params=pltpu.CompilerParams(needs_layout_passes=False,
                                             use_tc_tiling_on_sc=False),
        scratch_types=[
            pltpu.VMEM((nmacro_t, MACRO), jnp.int32),
            pltpu.VMEM((nmacro_t, MACRO), jnp.int32),
            pltpu.VMEM((ept,), jnp.float32),
            pltpu.VMEM((MACRO, d_pad), jnp.float32),
            pltpu.VMEM((MACRO, d_pad), jnp.float32),
            pltpu.VMEM((MACRO, F), jnp.float32),
            pltpu.SemaphoreType.DMA,
            pltpu.SemaphoreType.DMA,
            pltpu.VMEM_SHARED((n_pad, F), jnp.float32),
        ],
    )
    return fn(src2, dst2, q, k, zeros_nf, vt3)


def _tconv_sc(x, edge_index, p, name):
    src = edge_index[0].astype(jnp.int32)
    dst = edge_index[1].astype(jnp.int32)
    n, din = x.shape
    dout = p[name + '_Wq'].shape[0]
    Wcat = jnp.concatenate(
        [p[name + '_Wq'], p[name + '_Wk'], p[name + '_Wv'], p[name + '_Ws']],
        axis=0)
    bcat = jnp.concatenate(
        [p[name + '_bq'], p[name + '_bk'], p[name + '_bv'], p[name + '_bs']],
        axis=0)
    proj = _matmul_bias(x, Wcat, bcat)
    q = proj[:, 0 * dout:1 * dout]
    k = proj[:, 1 * dout:2 * dout]
    v = proj[:, 2 * dout:3 * dout]
    skip = proj[:, 3 * dout:4 * dout]

    n_pad = ((n + 127) // 128) * 128
    d_pad = ((dout + L - 1) // L) * L
    f_max = max(L, (SPMEM_BUDGET // (4 * n_pad)) // L * L)
    nch = -(-(dout + 1) // f_max)
    F = (-(-(dout + 1) // nch) + L - 1) // L * L

    qp = jnp.pad(q, ((0, n_pad - n), (0, d_pad - dout)))
    kp = jnp.pad(k, ((0, n_pad - n), (0, d_pad - dout)))
    vaug = jnp.concatenate([v, jnp.ones((n, 1), jnp.float32)], axis=1)
    vaug = jnp.pad(vaug, ((0, n_pad - n), (0, nch * F - (dout + 1))))
    vt3 = jnp.transpose(vaug.reshape(n_pad, nch, F), (1, 0, 2))
    zeros_nf = jnp.zeros((n_pad, F), jnp.float32)

    e_real = src.shape[0]
    nmacro_t = -(-e_real // (NW * MACRO))
    nmacro_t = ((nmacro_t + 7) // 8) * 8
    e_pad = NW * nmacro_t * MACRO
    src2 = jnp.pad(src, (0, e_pad - e_real)).reshape(-1, MACRO)
    dst2 = jnp.pad(dst, (0, e_pad - e_real)).reshape(-1, MACRO)

    out = _sc_edge_aggregate(src2, dst2, qp, kp, vt3, zeros_nf,
                             d=dout, e_real=e_real,
                             nmacro_t=nmacro_t, n_pad=n_pad)
    agg = out.sum(axis=1)
    agg = jnp.transpose(agg, (1, 0, 2)).reshape(n_pad, nch * F)
    num = agg[:n, :dout]
    den = agg[:n, dout]
    return num / (den[:, None] + 1e-16) + skip


def _gru_mean(x, p):
    def step(Wih, Whh, bih, bhh):
        gi = x @ Wih.T + bih
        gh = bhh
        i_r, i_z, i_n = jnp.split(gi, 3, axis=-1)
        h_r, h_z, h_n = jnp.split(gh, 3, axis=-1)
        r = jax.nn.sigmoid(i_r + h_r)
        z = jax.nn.sigmoid(i_z + h_z)
        nn = jnp.tanh(i_n + r * h_n)
        return (1.0 - z) * nn
    out = jnp.concatenate([
        step(p['gru_Wih_f'], p['gru_Whh_f'], p['gru_bih_f'], p['gru_bhh_f']),
        step(p['gru_Wih_b'], p['gru_Whh_b'], p['gru_bih_b'], p['gru_bhh_b']),
    ], axis=-1)
    out = jax.nn.relu(out)
    return jnp.mean(out, axis=0, keepdims=True)


def _set2set(x, p):
    nper = x.shape[0] // B
    d = x.shape[-1]
    xb = x.reshape(B, nper, d)
    q_star = jnp.zeros((B, 2 * d), dtype=x.dtype)
    h = jnp.zeros((B, d), dtype=x.dtype)
    c = jnp.zeros((B, d), dtype=x.dtype)
    for _ in range(2):
        g = (q_star @ p['lstm_Wih'].T + p['lstm_bih']
             + h @ p['lstm_Whh'].T + p['lstm_bhh'])
        ii, ff, gg, oo = jnp.split(g, 4, axis=-1)
        ii = jax.nn.sigmoid(ii)
        ff = jax.nn.sigmoid(ff)
        gg = jnp.tanh(gg)
        oo = jax.nn.sigmoid(oo)
        c = ff * c + ii * gg
        h = oo * jnp.tanh(c)
        q = h
        e = jnp.einsum('bnd,bd->bn', xb, q)
        emax = jnp.max(e, axis=1, keepdims=True)
        ex = jnp.exp(e - emax)
        a = ex / (jnp.sum(ex, axis=1, keepdims=True) + 1e-16)
        r = jnp.einsum('bn,bnd->bd', a, xb)
        q_star = jnp.concatenate([q, r], axis=-1)
    return q_star


def kernel(solute_adj, solute_meth, solvent_meth, solvent_adj_meth, smiles, params):
    p = params
    solute_smile = smiles[0]
    meth_solvent = smiles[5]
    sv = jnp.take(p['embed'], solute_smile, axis=0)
    mv = jnp.take(p['embed'], meth_solvent, axis=0)
    after_solute = jnp.tile(_gru_mean(sv, p), (B, 1))
    after_meth = jnp.tile(_gru_mean(mv, p), (B, 1))

    xs0 = solute_meth.reshape(B * N_SOLUTE, NFEAT)
    xv0 = solvent_meth.reshape(B * N_SOLVENT, NFEAT)
    init_s = _matmul_bias(xs0, p['fc1_W'], p['fc1_b'])
    init_v = _matmul_bias(xv0, p['fc1_W'], p['fc1_b'])

    xs = jax.nn.relu(_tconv_sc(xs0, solute_adj, p, 'c1'))
    xs = _tconv_sc(xs, solute_adj, p, 'c2') + init_s
    xv = jax.nn.relu(_tconv_sc(xv0, solvent_adj_meth, p, 'c1'))
    xv = _tconv_sc(xv, solvent_adj_meth, p, 'c2') + init_v

    ss = _set2set(xs, p)
    vv = _set2set(xv, p)
    data = jnp.concatenate([ss, after_solute, vv, after_meth], axis=1)
    data = jax.nn.relu(data @ p['fc2_W'].T + p['fc2_b'])
    data = jax.nn.relu(data @ p['fc3_W'].T + p['fc3_b'])
    data = jax.nn.relu(data @ p['fc4_W'].T + p['fc4_b'])
    return data @ p['fc5_W'].T + p['fc5_b']


# speedup vs baseline: 1.4841x; 1.4841x over previous
import functools
import math

import jax
import jax.numpy as jnp
from jax import lax
from jax.experimental import pallas as pl
from jax.experimental.pallas import tpu as pltpu
from jax.experimental.pallas import tpu_sc as plsc

B = 4
N_SOLUTE = 2076
N_SOLVENT = 16335
NFEAT = 128
NCLASS = 100

NC = 2
NS = 16
L = 16
NW = NC * NS

SPMEM_BUDGET = 4 * 2**20


def _mm_body(x_ref, w_ref, b_ref, o_ref):
    o_ref[...] = (
        jnp.dot(x_ref[...], w_ref[...], preferred_element_type=jnp.float32)
        + b_ref[...]
    )


def _matmul_bias(x, W, b, block_rows=512):
    n, din = x.shape
    dout = W.shape[0]
    n_pad = ((n + block_rows - 1) // block_rows) * block_rows
    if n_pad != n:
        x = jnp.pad(x, ((0, n_pad - n), (0, 0)))
    out = pl.pallas_call(
        _mm_body,
        grid=(n_pad // block_rows,),
        in_specs=[
            pl.BlockSpec((block_rows, din), lambda i: (i, 0)),
            pl.BlockSpec((din, dout), lambda i: (0, 0)),
            pl.BlockSpec((1, dout), lambda i: (0, 0)),
        ],
        out_specs=pl.BlockSpec((block_rows, dout), lambda i: (i, 0)),
        out_shape=jax.ShapeDtypeStruct((n_pad, dout), jnp.float32),
    )(x, W.T, b.reshape(1, dout))
    return out[:n]


def _edge_kernel_body(nch, d_pad, F, scale, e_real, nmacro_t, n_pad, MACRO,
                      zr, *refs):
    (src_hbm, dst_hbm, q_hbm, k_hbm, vt_hbm, out_hbm,
     src_v, dst_v, ex_v, q_rows, k_rows, v_rows, zbuf,
     sem1, sem2, sem3, agg_sp) = refs

    c = lax.axis_index("c")
    s = lax.axis_index("s")
    wid = s * NC + c
    mb = wid * nmacro_t
    rslice = n_pad // NS
    iota = lax.iota(jnp.int32, L)
    inv = jnp.float32(scale)

    pltpu.sync_copy(src_hbm.at[pl.ds(mb, nmacro_t)], src_v)
    pltpu.sync_copy(dst_hbm.at[pl.ds(mb, nmacro_t)], dst_v)
    last = nmacro_t - 1

    pltpu.async_copy(q_hbm.at[dst_v.at[0]], q_rows.at[0], sem1)
    pltpu.async_copy(k_hbm.at[src_v.at[0]], k_rows.at[0], sem2)

    def phase1(j, carry):
        p = lax.rem(j, 2)
        jn = jnp.minimum(j + 1, last)
        pltpu.async_copy(q_hbm.at[dst_v.at[jn]], q_rows.at[1 - p], sem1)
        pltpu.async_copy(k_hbm.at[src_v.at[jn]], k_rows.at[1 - p], sem2)
        pltpu.make_async_copy(q_hbm.at[dst_v.at[j]], q_rows.at[p],
                              sem1).wait()
        pltpu.make_async_copy(k_hbm.at[src_v.at[j]], k_rows.at[p],
                              sem2).wait()
        qr = q_rows.at[p]
        kr = k_rows.at[p]

        def dotq(qq, carry2):
            ev = jnp.zeros((L,), jnp.float32)
            for i16 in range(L):
                i = L * qq + i16
                acc = qr[i, pl.ds(0, L)] * kr[i, pl.ds(0, L)]
                for w in range(1, d_pad // L):
                    acc = acc + (qr[i, pl.ds(L * w, L)]
                                 * kr[i, pl.ds(L * w, L)])
                ev = jnp.where(iota == i16, jnp.sum(acc) * inv, ev)
            gid = (mb + j) * MACRO + L * qq + iota
            ex_v[pl.ds(j * MACRO + L * qq, L)] = jnp.where(
                gid < e_real, jnp.exp(ev), 0.0)
            return carry2
        lax.fori_loop(0, MACRO // L, dotq, 0)
        return carry
    lax.fori_loop(0, nmacro_t, phase1, 0)
    pe = lax.rem(last + 1, 2)
    pltpu.make_async_copy(q_hbm.at[dst_v.at[last]], q_rows.at[pe], sem1).wait()
    pltpu.make_async_copy(k_hbm.at[src_v.at[last]], k_rows.at[pe], sem2).wait()

    zvec = jnp.zeros((L,), jnp.float32)

    def zrow(r, carry):
        for w in range(F // L):
            zbuf[r, pl.ds(L * w, L)] = zvec
        return carry
    lax.fori_loop(0, zr, zrow, 0)

    def chunk_loop(ci, carry):
        def zcp(t, carry2):
            pltpu.sync_copy(zbuf, agg_sp.at[pl.ds(s * rslice + t * zr, zr)])
            return carry2
        lax.fori_loop(0, rslice // zr, zcp, 0)
        plsc.subcore_barrier()
        vt = vt_hbm.at[ci]
        pltpu.async_copy(vt.at[src_v.at[0]], v_rows.at[0], sem1)

        def phase2(j, carry2):
            p = lax.rem(j, 2)
            jn = jnp.minimum(j + 1, last)

            @pl.when(j > 0)
            def _():
                pltpu.make_async_copy(
                    v_rows.at[1 - p],
                    agg_sp.at[dst_v.at[jnp.maximum(j - 1, 0)]], sem3).wait()
            pltpu.async_copy(vt.at[src_v.at[jn]], v_rows.at[1 - p], sem1)
            pltpu.make_async_copy(vt.at[src_v.at[j]], v_rows.at[p],
                                  sem1).wait()
            vr = v_rows.at[p]

            def scaleq(qq, carry3):
                ex16 = ex_v[pl.ds(j * MACRO + L * qq, L)]
                for i16 in range(L):
                    i = L * qq + i16
                    exs = ex16[i16]
                    for w in range(F // L):
                        sl = pl.ds(L * w, L)
                        vr[i, sl] = vr[i, sl] * exs
                return carry3
            lax.fori_loop(0, MACRO // L, scaleq, 0)
            pltpu.async_copy(vr, agg_sp.at[dst_v.at[j]], sem3, add=True)
            return carry2
        lax.fori_loop(0, nmacro_t, phase2, 0)
        pltpu.make_async_copy(
            v_rows.at[lax.rem(last, 2)],
            agg_sp.at[dst_v.at[last]], sem3).wait()
        pltpu.make_async_copy(vt.at[src_v.at[last]],
                              v_rows.at[lax.rem(last + 1, 2)], sem1).wait()
        plsc.subcore_barrier()
        pltpu.sync_copy(agg_sp.at[pl.ds(s * rslice, rslice)],
                        out_hbm.at[ci].at[c].at[pl.ds(s * rslice, rslice)])
        return carry
    lax.fori_loop(0, nch, chunk_loop, 0)


def _zr_rows(rslice, F):
    best = 1
    for cand in range(1, rslice + 1):
        if rslice % cand == 0 and cand * F <= 2048:
            best = cand
    return best


def _sc_edge_aggregate(src2, dst2, q, k, vt3,
                       *, d, e_real, nmacro_t, n_pad, macro):
    nch, _, F = vt3.shape
    d_pad = q.shape[1]
    zr = _zr_rows(n_pad // NS, F)
    mesh = plsc.VectorSubcoreMesh(core_axis_name="c", subcore_axis_name="s",
                                  num_cores=NC, num_subcores=NS)
    ept = nmacro_t * macro
    body = functools.partial(_edge_kernel_body, nch, d_pad, F,
                             1.0 / math.sqrt(d), e_real, nmacro_t, n_pad,
                             macro, zr)
    fn = pl.kernel(
        body,
        out_type=jax.ShapeDtypeStruct((nch, NC, n_pad, F), jnp.float32),
        mesh=mesh,
        compiler_params=pltpu.CompilerParams(needs_layout_passes=False,
                                             use_tc_tiling_on_sc=False),
        scratch_types=[
            pltpu.VMEM((nmacro_t, macro), jnp.int32),
            pltpu.VMEM((nmacro_t, macro), jnp.int32),
            pltpu.VMEM((ept,), jnp.float32),
            pltpu.VMEM((2, macro, d_pad), jnp.float32),
            pltpu.VMEM((2, macro, d_pad), jnp.float32),
            pltpu.VMEM((2, macro, F), jnp.float32),
            pltpu.VMEM((zr, F), jnp.float32),
            pltpu.SemaphoreType.DMA,
            pltpu.SemaphoreType.DMA,
            pltpu.SemaphoreType.DMA,
            pltpu.VMEM_SHARED((n_pad, F), jnp.float32),
        ],
    )
    return fn(src2, dst2, q, k, vt3)


def _tconv_sc(x, edge_index, p, name):
    src = edge_index[0].astype(jnp.int32)
    dst = edge_index[1].astype(jnp.int32)
    n, din = x.shape
    dout = p[name + '_Wq'].shape[0]
    Wcat = jnp.concatenate(
        [p[name + '_Wq'], p[name + '_Wk'], p[name + '_Wv'], p[name + '_Ws']],
        axis=0)
    bcat = jnp.concatenate(
        [p[name + '_bq'], p[name + '_bk'], p[name + '_bv'], p[name + '_bs']],
        axis=0)
    proj = _matmul_bias(x, Wcat, bcat)
    q = proj[:, 0 * dout:1 * dout]
    k = proj[:, 1 * dout:2 * dout]
    v = proj[:, 2 * dout:3 * dout]
    skip = proj[:, 3 * dout:4 * dout]

    n_pad = ((n + 127) // 128) * 128
    d_pad = ((dout + L - 1) // L) * L
    f_max = max(L, (SPMEM_BUDGET // (4 * n_pad)) // L * L)
    nch = -(-(dout + 1) // f_max)
    F = (-(-(dout + 1) // nch) + L - 1) // L * L
    macro = 16 if (n_pad > 30000 and d_pad > 64) else 32

    qp = jnp.pad(q, ((0, n_pad - n), (0, d_pad - dout)))
    kp = jnp.pad(k, ((0, n_pad - n), (0, d_pad - dout)))
    vaug = jnp.concatenate([v, jnp.ones((n, 1), jnp.float32)], axis=1)
    vaug = jnp.pad(vaug, ((0, n_pad - n), (0, nch * F - (dout + 1))))
    vt3 = jnp.transpose(vaug.reshape(n_pad, nch, F), (1, 0, 2))

    e_real = src.shape[0]
    nmacro_t = -(-e_real // (NW * macro))
    nmacro_t = ((nmacro_t + 7) // 8) * 8
    e_pad = NW * nmacro_t * macro
    src2 = jnp.pad(src, (0, e_pad - e_real)).reshape(-1, macro)
    dst2 = jnp.pad(dst, (0, e_pad - e_real)).reshape(-1, macro)

    out = _sc_edge_aggregate(src2, dst2, qp, kp, vt3,
                             d=dout, e_real=e_real,
                             nmacro_t=nmacro_t, n_pad=n_pad, macro=macro)
    agg = out.sum(axis=1)
    agg = jnp.transpose(agg, (1, 0, 2)).reshape(n_pad, nch * F)
    num = agg[:n, :dout]
    den = agg[:n, dout]
    return num / (den[:, None] + 1e-16) + skip


def _gru_mean(x, p):
    def step(Wih, Whh, bih, bhh):
        gi = x @ Wih.T + bih
        gh = bhh
        i_r, i_z, i_n = jnp.split(gi, 3, axis=-1)
        h_r, h_z, h_n = jnp.split(gh, 3, axis=-1)
        r = jax.nn.sigmoid(i_r + h_r)
        z = jax.nn.sigmoid(i_z + h_z)
        nn = jnp.tanh(i_n + r * h_n)
        return (1.0 - z) * nn
    out = jnp.concatenate([
        step(p['gru_Wih_f'], p['gru_Whh_f'], p['gru_bih_f'], p['gru_bhh_f']),
        step(p['gru_Wih_b'], p['gru_Whh_b'], p['gru_bih_b'], p['gru_bhh_b']),
    ], axis=-1)
    out = jax.nn.relu(out)
    return jnp.mean(out, axis=0, keepdims=True)


def _set2set(x, p):
    nper = x.shape[0] // B
    d = x.shape[-1]
    xb = x.reshape(B, nper, d)
    q_star = jnp.zeros((B, 2 * d), dtype=x.dtype)
    h = jnp.zeros((B, d), dtype=x.dtype)
    c = jnp.zeros((B, d), dtype=x.dtype)
    for _ in range(2):
        g = (q_star @ p['lstm_Wih'].T + p['lstm_bih']
             + h @ p['lstm_Whh'].T + p['lstm_bhh'])
        ii, ff, gg, oo = jnp.split(g, 4, axis=-1)
        ii = jax.nn.sigmoid(ii)
        ff = jax.nn.sigmoid(ff)
        gg = jnp.tanh(gg)
        oo = jax.nn.sigmoid(oo)
        c = ff * c + ii * gg
        h = oo * jnp.tanh(c)
        q = h
        e = jnp.einsum('bnd,bd->bn', xb, q)
        emax = jnp.max(e, axis=1, keepdims=True)
        ex = jnp.exp(e - emax)
        a = ex / (jnp.sum(ex, axis=1, keepdims=True) + 1e-16)
        r = jnp.einsum('bn,bnd->bd', a, xb)
        q_star = jnp.concatenate([q, r], axis=-1)
    return q_star


def kernel(solute_adj, solute_meth, solvent_meth, solvent_adj_meth, smiles, params):
    p = params
    solute_smile = smiles[0]
    meth_solvent = smiles[5]
    sv = jnp.take(p['embed'], solute_smile, axis=0)
    mv = jnp.take(p['embed'], meth_solvent, axis=0)
    after_solute = jnp.tile(_gru_mean(sv, p), (B, 1))
    after_meth = jnp.tile(_gru_mean(mv, p), (B, 1))

    xs0 = solute_meth.reshape(B * N_SOLUTE, NFEAT)
    xv0 = solvent_meth.reshape(B * N_SOLVENT, NFEAT)
    init_s = _matmul_bias(xs0, p['fc1_W'], p['fc1_b'])
    init_v = _matmul_bias(xv0, p['fc1_W'], p['fc1_b'])

    xv = jax.nn.relu(_tconv_sc(xv0, solvent_adj_meth, p, 'c1'))
    xv = _tconv_sc(xv, solvent_adj_meth, p, 'c2') + init_v
    xs0b, xv = lax.optimization_barrier((xs0, xv))
    xs = jax.nn.relu(_tconv_sc(xs0b, solute_adj, p, 'c1'))
    xs = _tconv_sc(xs, solute_adj, p, 'c2') + init_s

    ss = _set2set(xs, p)
    vv = _set2set(xv, p)
    data = jnp.concatenate([ss, after_solute, vv, after_meth], axis=1)
    data = jax.nn.relu(data @ p['fc2_W'].T + p['fc2_b'])
    data = jax.nn.relu(data @ p['fc3_W'].T + p['fc3_b'])
    data = jax.nn.relu(data @ p['fc4_W'].T + p['fc4_b'])
    return data @ p['fc5_W'].T + p['fc5_b']


# speedup vs baseline: 1.8050x; 1.2162x over previous
import functools
import math

import jax
import jax.numpy as jnp
from jax import lax
from jax.experimental import pallas as pl
from jax.experimental.pallas import tpu as pltpu
from jax.experimental.pallas import tpu_sc as plsc

B = 4
N_SOLUTE = 2076
N_SOLVENT = 16335
NFEAT = 128
NCLASS = 100

NC = 2
NS = 16
L = 16
NW = NC * NS

SPMEM_BUDGET = 4 * 2**20


def _mm_body(x_ref, w_ref, b_ref, o_ref):
    o_ref[...] = (
        jnp.dot(x_ref[...], w_ref[...], preferred_element_type=jnp.float32)
        + b_ref[...]
    )


def _matmul_bias(x, W, b, block_rows=512):
    n, din = x.shape
    dout = W.shape[0]
    n_pad = ((n + block_rows - 1) // block_rows) * block_rows
    if n_pad != n:
        x = jnp.pad(x, ((0, n_pad - n), (0, 0)))
    out = pl.pallas_call(
        _mm_body,
        grid=(n_pad // block_rows,),
        in_specs=[
            pl.BlockSpec((block_rows, din), lambda i: (i, 0)),
            pl.BlockSpec((din, dout), lambda i: (0, 0)),
            pl.BlockSpec((1, dout), lambda i: (0, 0)),
        ],
        out_specs=pl.BlockSpec((block_rows, dout), lambda i: (i, 0)),
        out_shape=jax.ShapeDtypeStruct((n_pad, dout), jnp.float32),
    )(x, W.T, b.reshape(1, dout))
    return out[:n]


def _edge_kernel_body(nch, d_pad, F, scale, e_real, nmacro_t, n_pad, MACRO,
                      zr, *refs):
    (src_hbm, dst_hbm, q_hbm, k_hbm, vt_hbm, out_hbm,
     src_v, dst_v, ex_v, q_rows, k_rows, v_rows, zbuf,
     sem1, sem2, sem3, agg_sp) = refs

    c = lax.axis_index("c")
    s = lax.axis_index("s")
    wid = s * NC + c
    mb = wid * nmacro_t
    rslice = n_pad // NS
    iota = lax.iota(jnp.int32, L)
    inv = jnp.float32(scale)

    pltpu.sync_copy(src_hbm.at[pl.ds(mb, nmacro_t)], src_v)
    pltpu.sync_copy(dst_hbm.at[pl.ds(mb, nmacro_t)], dst_v)
    last = nmacro_t - 1

    pltpu.async_copy(q_hbm.at[dst_v.at[0]], q_rows.at[0], sem1)
    pltpu.async_copy(k_hbm.at[src_v.at[0]], k_rows.at[0], sem2)

    def phase1(j, carry):
        p = lax.rem(j, 2)
        jn = jnp.minimum(j + 1, last)
        pltpu.async_copy(q_hbm.at[dst_v.at[jn]], q_rows.at[1 - p], sem1)
        pltpu.async_copy(k_hbm.at[src_v.at[jn]], k_rows.at[1 - p], sem2)
        pltpu.make_async_copy(q_hbm.at[dst_v.at[j]], q_rows.at[p],
                              sem1).wait()
        pltpu.make_async_copy(k_hbm.at[src_v.at[j]], k_rows.at[p],
                              sem2).wait()
        qr = q_rows.at[p]
        kr = k_rows.at[p]

        def dotq(qq, carry2):
            ev = jnp.zeros((L,), jnp.float32)
            for i16 in range(L):
                i = L * qq + i16
                acc = qr[i, pl.ds(0, L)] * kr[i, pl.ds(0, L)]
                for w in range(1, d_pad // L):
                    acc = acc + (qr[i, pl.ds(L * w, L)]
                                 * kr[i, pl.ds(L * w, L)])
                ev = jnp.where(iota == i16, jnp.sum(acc) * inv, ev)
            gid = (mb + j) * MACRO + L * qq + iota
            ex_v[pl.ds(j * MACRO + L * qq, L)] = jnp.where(
                gid < e_real, jnp.exp(ev), 0.0)
            return carry2
        lax.fori_loop(0, MACRO // L, dotq, 0)
        return carry
    lax.fori_loop(0, nmacro_t, phase1, 0)
    pe = lax.rem(last + 1, 2)
    pltpu.make_async_copy(q_hbm.at[dst_v.at[last]], q_rows.at[pe], sem1).wait()
    pltpu.make_async_copy(k_hbm.at[src_v.at[last]], k_rows.at[pe], sem2).wait()

    zvec = jnp.zeros((L,), jnp.float32)

    def zrow(r, carry):
        for w in range(F // L):
            zbuf[r, pl.ds(L * w, L)] = zvec
        return carry
    lax.fori_loop(0, zr, zrow, 0)

    def chunk_loop(ci, carry):
        def zcp(t, carry2):
            pltpu.sync_copy(zbuf, agg_sp.at[pl.ds(s * rslice + t * zr, zr)])
            return carry2
        lax.fori_loop(0, rslice // zr, zcp, 0)
        plsc.subcore_barrier()
        vt = vt_hbm.at[ci]
        for k in range(3):
            pltpu.async_copy(vt.at[src_v.at[min(k, 7)]], v_rows.at[k], sem1)

        def phase2(j, carry2):
            p = lax.rem(j, 4)
            pn = lax.rem(j + 3, 4)
            jn = jnp.minimum(j + 3, last)

            @pl.when(j > 0)
            def _():
                pltpu.make_async_copy(
                    v_rows.at[pn],
                    agg_sp.at[dst_v.at[jnp.maximum(j - 1, 0)]], sem3).wait()
            pltpu.async_copy(vt.at[src_v.at[jn]], v_rows.at[pn], sem1)
            pltpu.make_async_copy(vt.at[src_v.at[j]], v_rows.at[p],
                                  sem1).wait()
            vr = v_rows.at[p]

            def scaleq(qq, carry3):
                ex16 = ex_v[pl.ds(j * MACRO + L * qq, L)]
                for i16 in range(L):
                    i = L * qq + i16
                    exs = ex16[i16]
                    for w in range(F // L):
                        sl = pl.ds(L * w, L)
                        vr[i, sl] = vr[i, sl] * exs
                return carry3
            lax.fori_loop(0, MACRO // L, scaleq, 0)
            pltpu.async_copy(vr, agg_sp.at[dst_v.at[j]], sem3, add=True)
            return carry2
        lax.fori_loop(0, nmacro_t, phase2, 0)
        pltpu.make_async_copy(
            v_rows.at[lax.rem(last, 4)],
            agg_sp.at[dst_v.at[last]], sem3).wait()
        for k in range(1, 4):
            pltpu.make_async_copy(vt.at[src_v.at[last]],
                                  v_rows.at[lax.rem(last + k, 4)],
                                  sem1).wait()
        plsc.subcore_barrier()
        pltpu.sync_copy(agg_sp.at[pl.ds(s * rslice, rslice)],
                        out_hbm.at[ci].at[c].at[pl.ds(s * rslice, rslice)])
        return carry
    lax.fori_loop(0, nch, chunk_loop, 0)


def _zr_rows(rslice, F):
    best = 1
    for cand in range(1, rslice + 1):
        if rslice % cand == 0 and cand * F <= 2048:
            best = cand
    return best


def _sc_edge_aggregate(src2, dst2, q, k, vt3,
                       *, d, e_real, nmacro_t, n_pad, macro):
    nch, _, F = vt3.shape
    d_pad = q.shape[1]
    zr = _zr_rows(n_pad // NS, F)
    mesh = plsc.VectorSubcoreMesh(core_axis_name="c", subcore_axis_name="s",
                                  num_cores=NC, num_subcores=NS)
    ept = nmacro_t * macro
    body = functools.partial(_edge_kernel_body, nch, d_pad, F,
                             1.0 / math.sqrt(d), e_real, nmacro_t, n_pad,
                             macro, zr)
    fn = pl.kernel(
        body,
        out_type=jax.ShapeDtypeStruct((nch, NC, n_pad, F), jnp.float32),
        mesh=mesh,
        compiler_params=pltpu.CompilerParams(needs_layout_passes=False,
                                             use_tc_tiling_on_sc=False),
        scratch_types=[
            pltpu.VMEM((nmacro_t, macro), jnp.int32),
            pltpu.VMEM((nmacro_t, macro), jnp.int32),
            pltpu.VMEM((ept,), jnp.float32),
            pltpu.VMEM((2, macro, d_pad), jnp.float32),
            pltpu.VMEM((2, macro, d_pad), jnp.float32),
            pltpu.VMEM((4, macro, F), jnp.float32),
            pltpu.VMEM((zr, F), jnp.float32),
            pltpu.SemaphoreType.DMA,
            pltpu.SemaphoreType.DMA,
            pltpu.SemaphoreType.DMA,
            pltpu.VMEM_SHARED((n_pad, F), jnp.float32),
        ],
    )
    return fn(src2, dst2, q, k, vt3)


def _tconv_sc(x, edge_index, p, name):
    src = edge_index[0].astype(jnp.int32)
    dst = edge_index[1].astype(jnp.int32)
    n, din = x.shape
    dout = p[name + '_Wq'].shape[0]
    Wcat = jnp.concatenate(
        [p[name + '_Wq'], p[name + '_Wk'], p[name + '_Wv'], p[name + '_Ws']],
        axis=0)
    bcat = jnp.concatenate(
        [p[name + '_bq'], p[name + '_bk'], p[name + '_bv'], p[name + '_bs']],
        axis=0)
    proj = _matmul_bias(x, Wcat, bcat)
    q = proj[:, 0 * dout:1 * dout]
    k = proj[:, 1 * dout:2 * dout]
    v = proj[:, 2 * dout:3 * dout]
    skip = proj[:, 3 * dout:4 * dout]

    n_pad = ((n + 127) // 128) * 128
    d_pad = ((dout + L - 1) // L) * L
    f_max = max(L, (SPMEM_BUDGET // (4 * n_pad)) // L * L)
    nch = -(-(dout + 1) // f_max)
    F = (-(-(dout + 1) // nch) + L - 1) // L * L
    macro = 16 if (n_pad > 30000 and d_pad > 64) else 32

    qp = jnp.pad(q, ((0, n_pad - n), (0, d_pad - dout)))
    kp = jnp.pad(k, ((0, n_pad - n), (0, d_pad - dout)))
    vaug = jnp.concatenate([v, jnp.ones((n, 1), jnp.float32)], axis=1)
    vaug = jnp.pad(vaug, ((0, n_pad - n), (0, nch * F - (dout + 1))))
    vt3 = jnp.transpose(vaug.reshape(n_pad, nch, F), (1, 0, 2))

    e_real = src.shape[0]
    nmacro_t = -(-e_real // (NW * macro))
    nmacro_t = ((nmacro_t + 7) // 8) * 8
    e_pad = NW * nmacro_t * macro
    src2 = jnp.pad(src, (0, e_pad - e_real)).reshape(-1, macro)
    dst2 = jnp.pad(dst, (0, e_pad - e_real)).reshape(-1, macro)

    out = _sc_edge_aggregate(src2, dst2, qp, kp, vt3,
                             d=dout, e_real=e_real,
                             nmacro_t=nmacro_t, n_pad=n_pad, macro=macro)
    agg = out.sum(axis=1)
    agg = jnp.transpose(agg, (1, 0, 2)).reshape(n_pad, nch * F)
    num = agg[:n, :dout]
    den = agg[:n, dout]
    return num / (den[:, None] + 1e-16) + skip


def _gru_mean(x, p):
    def step(Wih, Whh, bih, bhh):
        gi = x @ Wih.T + bih
        gh = bhh
        i_r, i_z, i_n = jnp.split(gi, 3, axis=-1)
        h_r, h_z, h_n = jnp.split(gh, 3, axis=-1)
        r = jax.nn.sigmoid(i_r + h_r)
        z = jax.nn.sigmoid(i_z + h_z)
        nn = jnp.tanh(i_n + r * h_n)
        return (1.0 - z) * nn
    out = jnp.concatenate([
        step(p['gru_Wih_f'], p['gru_Whh_f'], p['gru_bih_f'], p['gru_bhh_f']),
        step(p['gru_Wih_b'], p['gru_Whh_b'], p['gru_bih_b'], p['gru_bhh_b']),
    ], axis=-1)
    out = jax.nn.relu(out)
    return jnp.mean(out, axis=0, keepdims=True)


def _set2set(x, p):
    nper = x.shape[0] // B
    d = x.shape[-1]
    xb = x.reshape(B, nper, d)
    q_star = jnp.zeros((B, 2 * d), dtype=x.dtype)
    h = jnp.zeros((B, d), dtype=x.dtype)
    c = jnp.zeros((B, d), dtype=x.dtype)
    for _ in range(2):
        g = (q_star @ p['lstm_Wih'].T + p['lstm_bih']
             + h @ p['lstm_Whh'].T + p['lstm_bhh'])
        ii, ff, gg, oo = jnp.split(g, 4, axis=-1)
        ii = jax.nn.sigmoid(ii)
        ff = jax.nn.sigmoid(ff)
        gg = jnp.tanh(gg)
        oo = jax.nn.sigmoid(oo)
        c = ff * c + ii * gg
        h = oo * jnp.tanh(c)
        q = h
        e = jnp.einsum('bnd,bd->bn', xb, q)
        emax = jnp.max(e, axis=1, keepdims=True)
        ex = jnp.exp(e - emax)
        a = ex / (jnp.sum(ex, axis=1, keepdims=True) + 1e-16)
        r = jnp.einsum('bn,bnd->bd', a, xb)
        q_star = jnp.concatenate([q, r], axis=-1)
    return q_star


def kernel(solute_adj, solute_meth, solvent_meth, solvent_adj_meth, smiles, params):
    p = params
    solute_smile = smiles[0]
    meth_solvent = smiles[5]
    sv = jnp.take(p['embed'], solute_smile, axis=0)
    mv = jnp.take(p['embed'], meth_solvent, axis=0)
    after_solute = jnp.tile(_gru_mean(sv, p), (B, 1))
    after_meth = jnp.tile(_gru_mean(mv, p), (B, 1))

    xs0 = solute_meth.reshape(B * N_SOLUTE, NFEAT)
    xv0 = solvent_meth.reshape(B * N_SOLVENT, NFEAT)
    init_s = _matmul_bias(xs0, p['fc1_W'], p['fc1_b'])
    init_v = _matmul_bias(xv0, p['fc1_W'], p['fc1_b'])

    xv = jax.nn.relu(_tconv_sc(xv0, solvent_adj_meth, p, 'c1'))
    xv = _tconv_sc(xv, solvent_adj_meth, p, 'c2') + init_v
    xs0b, xv = lax.optimization_barrier((xs0, xv))
    xs = jax.nn.relu(_tconv_sc(xs0b, solute_adj, p, 'c1'))
    xs = _tconv_sc(xs, solute_adj, p, 'c2') + init_s

    ss = _set2set(xs, p)
    vv = _set2set(xv, p)
    data = jnp.concatenate([ss, after_solute, vv, after_meth], axis=1)
    data = jax.nn.relu(data @ p['fc2_W'].T + p['fc2_b'])
    data = jax.nn.relu(data @ p['fc3_W'].T + p['fc3_b'])
    data = jax.nn.relu(data @ p['fc4_W'].T + p['fc4_b'])
    return data @ p['fc5_W'].T + p['fc5_b']
